# baseline (speedup 1.0000x reference)
import functools

import jax
import jax.numpy as jnp
from jax import lax
from jax.experimental import pallas as pl
from jax.experimental.pallas import tpu as pltpu
from jax.experimental.pallas import tpu_sc as plsc

D = 24
NBK = 1024


def kernel(product_id, popular_score, product_category,
           title_table, scores_table, category_table, buckets):
    B = product_id.shape[0]
    VS, _ = scores_table.shape
    VC, _ = category_table.shape
    NBUCK = buckets.shape[0]
    info = plsc.get_sparse_core_info()
    NC, NS, L = info.num_cores, info.num_subcores, info.num_lanes
    NW = NC * NS
    bpw = B // NW
    nrow = bpw // 128
    nchunk = bpw // L

    pid3 = product_id.astype(jnp.int32).reshape(NW, nrow, 128)
    cat2 = product_category.astype(jnp.int32).reshape(NW, bpw)
    ps2 = popular_score.reshape(NW, bpw)

    mesh = plsc.VectorSubcoreMesh(core_axis_name="c", subcore_axis_name="s")

    @functools.partial(
        pl.kernel, mesh=mesh,
        compiler_params=pltpu.CompilerParams(
            needs_layout_passes=False, use_tc_tiling_on_sc=False),
        out_type=jax.ShapeDtypeStruct((3 * D, B), jnp.float32),
        scratch_types=[
            pltpu.VMEM((nrow, 128), jnp.int32),
            pltpu.VMEM((bpw,), jnp.int32),
            pltpu.VMEM((bpw,), jnp.float32),
            pltpu.VMEM((NBK,), jnp.float32),
            pltpu.VMEM((bpw, D), jnp.float32),
            pltpu.VMEM((VS, D), jnp.float32),
            pltpu.VMEM((VC, D), jnp.float32),
            pltpu.VMEM((3 * D, bpw), jnp.float32),
            pltpu.SemaphoreType.DMA,
        ],
    )
    def sc_kernel(pid_hbm, ps_hbm, cat_hbm, ttab, stab, ctab, bkt_hbm,
                  out_hbm, pid_v, cat_v, ps_v, bkt_v, trows, stab_v,
                  ctab_v, outt_v, sem_t):
        wid = lax.axis_index("s") * NC + lax.axis_index("c")
        base = wid * bpw

        pltpu.sync_copy(pid_hbm.at[wid], pid_v)
        copies = []
        for j in range(nrow):
            copies.append(pltpu.async_copy(
                ttab.at[pid_v.at[j]], trows.at[pl.ds(j * 128, 128)], sem_t))

        pltpu.sync_copy(cat_hbm.at[wid], cat_v)
        pltpu.sync_copy(ps_hbm.at[wid], ps_v)
        pltpu.sync_copy(stab, stab_v)
        pltpu.sync_copy(ctab, ctab_v)
        pltpu.sync_copy(bkt_hbm, bkt_v.at[pl.ds(0, NBUCK)])
        sentinel = jnp.full((L,), 2.0, jnp.float32)
        bkt_v[pl.ds(NBUCK, L)] = sentinel
        bkt_v[pl.ds(NBK - L, L)] = sentinel

        for c in copies:
            c.wait()

        iota = lax.iota(jnp.int32, L)

        def chunk(i, carry):
            off = pl.multiple_of(i * L, L)
            v = ps_v[pl.ds(off, L)]
            pos = jnp.zeros((L,), jnp.int32)
            step = NBK // 2
            while step >= 1:
                probe = pos + (step - 1)
                bv = plsc.load_gather(bkt_v, [probe])
                pos = jnp.where(bv <= v, pos + step, pos)
                step //= 2
            items = off + iota
            cats = cat_v[pl.ds(off, L)]
            for c in range(D):
                cvec = jnp.full((L,), c, jnp.int32)
                outt_v[c, pl.ds(off, L)] = plsc.load_gather(
                    trows, [items, cvec])
                outt_v[D + c, pl.ds(off, L)] = plsc.load_gather(
                    stab_v, [pos, cvec])
                outt_v[2 * D + c, pl.ds(off, L)] = plsc.load_gather(
                    ctab_v, [cats, cvec])
            return carry

        lax.fori_loop(0, nchunk, chunk, 0)

        pltpu.sync_copy(outt_v, out_hbm.at[:, pl.ds(base, bpw)])

    out_t = sc_kernel(pid3, ps2, cat2, title_table, scores_table,
                      category_table, buckets)
    return out_t.T


# speedup vs baseline: 1.5590x; 1.5590x over previous
import functools

import jax
import jax.numpy as jnp
from jax import lax
from jax.experimental import pallas as pl
from jax.experimental.pallas import tpu as pltpu
from jax.experimental.pallas import tpu_sc as plsc

D = 24
NBK = 1024


def kernel(product_id, popular_score, product_category,
           title_table, scores_table, category_table, buckets):
    B = product_id.shape[0]
    VT = title_table.shape[0]
    VS = scores_table.shape[0]
    VC = category_table.shape[0]
    NBUCK = buckets.shape[0]
    info = plsc.get_sparse_core_info()
    NC, NS, L = info.num_cores, info.num_subcores, info.num_lanes
    NW = NC * NS
    bpw = B // NW
    nrow = bpw // 128
    nchunk = bpw // L

    tt1 = title_table.T.reshape(D * VT)
    st1 = scores_table.T.reshape(D * VS)
    ct1 = category_table.T.reshape(D * VC)
    pid2 = product_id.astype(jnp.int32).reshape(NW, bpw)
    cat2 = product_category.astype(jnp.int32).reshape(NW, bpw)
    ps2 = popular_score.reshape(NW, bpw)

    mesh = plsc.VectorSubcoreMesh(core_axis_name="c", subcore_axis_name="s")

    @functools.partial(
        pl.kernel, mesh=mesh,
        compiler_params=pltpu.CompilerParams(
            needs_layout_passes=False, use_tc_tiling_on_sc=False),
        out_type=jax.ShapeDtypeStruct((3 * D, B), jnp.float32),
        scratch_types=[
            pltpu.VMEM((bpw,), jnp.int32),
            pltpu.VMEM((bpw,), jnp.int32),
            pltpu.VMEM((bpw,), jnp.float32),
            pltpu.VMEM((NBK,), jnp.float32),
            pltpu.VMEM((D * nrow, 128), jnp.int32),
            pltpu.VMEM((D * VS,), jnp.float32),
            pltpu.VMEM((D * VC,), jnp.float32),
            pltpu.VMEM((3 * D, bpw), jnp.float32),
            pltpu.SemaphoreType.DMA,
        ],
    )
    def sc_kernel(pid_hbm, ps_hbm, cat_hbm, ttab, stab, ctab, bkt_hbm,
                  out_hbm, pid_v, cat_v, ps_v, bkt_v, tidx_v, stab_v,
                  ctab_v, outt_v, sem_t):
        wid = lax.axis_index("s") * NC + lax.axis_index("c")
        base = wid * bpw

        pltpu.sync_copy(pid_hbm.at[wid], pid_v)

        def build(i, carry):
            off = pl.multiple_of(i * L, L)
            pid16 = pid_v[pl.ds(off, L)]
            row0 = i // 8
            col = pl.multiple_of((i % 8) * L, L)
            for c in range(D):
                tidx_v[c * nrow + row0, pl.ds(col, L)] = pid16 + (c * VT)
            return carry

        lax.fori_loop(0, nchunk, build, 0)

        copies = []
        for c in range(D):
            for j in range(nrow):
                copies.append(pltpu.async_copy(
                    ttab.at[tidx_v.at[c * nrow + j]],
                    outt_v.at[c, pl.ds(j * 128, 128)], sem_t))

        pltpu.sync_copy(cat_hbm.at[wid], cat_v)
        pltpu.sync_copy(ps_hbm.at[wid], ps_v)
        pltpu.sync_copy(stab, stab_v)
        pltpu.sync_copy(ctab, ctab_v)
        pltpu.sync_copy(bkt_hbm, bkt_v.at[pl.ds(0, NBUCK)])
        sentinel = jnp.full((L,), 2.0, jnp.float32)
        bkt_v[pl.ds(NBUCK, L)] = sentinel
        bkt_v[pl.ds(NBK - L, L)] = sentinel

        def chunk(i, carry):
            off = pl.multiple_of(i * L, L)
            v = ps_v[pl.ds(off, L)]
            pos = jnp.zeros((L,), jnp.int32)
            step = NBK // 2
            while step >= 1:
                probe = pos + (step - 1)
                bv = plsc.load_gather(bkt_v, [probe])
                pos = jnp.where(bv <= v, pos + step, pos)
                step //= 2
            cats = cat_v[pl.ds(off, L)]
            for c in range(D):
                outt_v[D + c, pl.ds(off, L)] = plsc.load_gather(
                    stab_v, [pos + (c * VS)])
                outt_v[2 * D + c, pl.ds(off, L)] = plsc.load_gather(
                    ctab_v, [cats + (c * VC)])
            return carry

        lax.fori_loop(0, nchunk, chunk, 0)

        for c in copies:
            c.wait()

        pltpu.sync_copy(outt_v, out_hbm.at[:, pl.ds(base, bpw)])

    out_t = sc_kernel(pid2, ps2, cat2, tt1, st1, ct1, buckets)
    return out_t.T


# speedup vs baseline: 1.7128x; 1.0986x over previous
import functools

import jax
import jax.numpy as jnp
from jax import lax
from jax.experimental import pallas as pl
from jax.experimental.pallas import tpu as pltpu
from jax.experimental.pallas import tpu_sc as plsc

D = 24
NBK = 1024


def kernel(product_id, popular_score, product_category,
           title_table, scores_table, category_table, buckets):
    B = product_id.shape[0]
    VT = title_table.shape[0]
    VS = scores_table.shape[0]
    VC = category_table.shape[0]
    NBUCK = buckets.shape[0]
    info = plsc.get_sparse_core_info()
    NC, NS, L = info.num_cores, info.num_subcores, info.num_lanes
    NW = NC * NS
    bpw = B // NW
    nrow = bpw // 128
    nchunk = bpw // L

    tt1 = title_table.T.reshape(D * VT)
    st1 = scores_table.T.reshape(D * VS)
    ct1 = category_table.T.reshape(D * VC)
    pid2 = product_id.astype(jnp.int32).reshape(NW, bpw)
    cat2 = product_category.astype(jnp.int32).reshape(NW, bpw)
    ps2 = popular_score.reshape(NW, bpw)

    mesh = plsc.VectorSubcoreMesh(core_axis_name="c", subcore_axis_name="s")

    @functools.partial(
        pl.kernel, mesh=mesh,
        compiler_params=pltpu.CompilerParams(
            needs_layout_passes=False, use_tc_tiling_on_sc=False),
        out_type=jax.ShapeDtypeStruct((3 * D // 8, B // 128, 8, 128),
                                      jnp.float32),
        scratch_types=[
            pltpu.VMEM((bpw,), jnp.int32),
            pltpu.VMEM((bpw,), jnp.int32),
            pltpu.VMEM((bpw,), jnp.float32),
            pltpu.VMEM((NBK,), jnp.float32),
            pltpu.VMEM((D * nrow, 128), jnp.int32),
            pltpu.VMEM((D * VS,), jnp.float32),
            pltpu.VMEM((D * VC,), jnp.float32),
            pltpu.VMEM((3 * D // 8, nrow, 8, 128), jnp.float32),
            pltpu.SemaphoreType.DMA,
        ],
    )
    def sc_kernel(pid_hbm, ps_hbm, cat_hbm, ttab, stab, ctab, bkt_hbm,
                  out_hbm, pid_v, cat_v, ps_v, bkt_v, tidx_v, stab_v,
                  ctab_v, outt_v, sem_t):
        wid = lax.axis_index("s") * NC + lax.axis_index("c")
        base = wid * bpw

        pltpu.sync_copy(pid_hbm.at[wid], pid_v)

        def build(i, carry):
            off = pl.multiple_of(i * L, L)
            pid16 = pid_v[pl.ds(off, L)]
            row0 = i // 8
            col = pl.multiple_of((i % 8) * L, L)
            for c in range(D):
                tidx_v[c * nrow + row0, pl.ds(col, L)] = pid16 + (c * VT)
            return carry

        lax.fori_loop(0, nchunk, build, 0)

        copies = []
        for c in range(D):
            for j in range(nrow):
                copies.append(pltpu.async_copy(
                    ttab.at[tidx_v.at[c * nrow + j]],
                    outt_v.at[c // 8, j, c % 8], sem_t))

        pltpu.sync_copy(cat_hbm.at[wid], cat_v)
        pltpu.sync_copy(ps_hbm.at[wid], ps_v)
        pltpu.sync_copy(stab, stab_v)
        pltpu.sync_copy(ctab, ctab_v)
        pltpu.sync_copy(bkt_hbm, bkt_v.at[pl.ds(0, NBUCK)])
        sentinel = jnp.full((L,), 2.0, jnp.float32)
        bkt_v[pl.ds(NBUCK, L)] = sentinel
        bkt_v[pl.ds(NBK - L, L)] = sentinel

        def chunk(i, carry):
            off = pl.multiple_of(i * L, L)
            v = ps_v[pl.ds(off, L)]
            pos = jnp.zeros((L,), jnp.int32)
            step = NBK // 2
            while step >= 1:
                probe = pos + (step - 1)
                bv = plsc.load_gather(bkt_v, [probe])
                pos = jnp.where(bv <= v, pos + step, pos)
                step //= 2
            cats = cat_v[pl.ds(off, L)]
            bc = i // 8
            cc = pl.multiple_of((i % 8) * L, L)
            for c in range(D):
                ds_ = D + c
                dc = 2 * D + c
                outt_v[ds_ // 8, bc, ds_ % 8, pl.ds(cc, L)] = \
                    plsc.load_gather(stab_v, [pos + (c * VS)])
                outt_v[dc // 8, bc, dc % 8, pl.ds(cc, L)] = \
                    plsc.load_gather(ctab_v, [cats + (c * VC)])
            return carry

        lax.fori_loop(0, nchunk, chunk, 0)

        for c in copies:
            c.wait()

        pltpu.sync_copy(outt_v, out_hbm.at[:, pl.ds(wid * nrow, nrow)])

    out4 = sc_kernel(pid2, ps2, cat2, tt1, st1, ct1, buckets)
    return out4.transpose(0, 2, 1, 3).reshape(3 * D, B).T


# speedup vs baseline: 1.9088x; 1.1144x over previous
import functools

import jax
import jax.numpy as jnp
from jax import lax
from jax.experimental import pallas as pl
from jax.experimental.pallas import tpu as pltpu
from jax.experimental.pallas import tpu_sc as plsc

D = 24
NBK = 1024


def kernel(product_id, popular_score, product_category,
           title_table, scores_table, category_table, buckets):
    B = product_id.shape[0]
    VT = title_table.shape[0]
    VS = scores_table.shape[0]
    VC = category_table.shape[0]
    NBUCK = buckets.shape[0]
    info = plsc.get_sparse_core_info()
    NC, NS, L = info.num_cores, info.num_subcores, info.num_lanes
    NW = NC * NS
    bpw = B // NW
    nrow = bpw // 128
    nchunk = bpw // L

    tt1 = title_table.T.reshape(D * VT)
    st1 = scores_table.T.reshape(D * VS)
    ct1 = category_table.T.reshape(D * VC)
    pid2 = product_id.astype(jnp.int32).reshape(NW, bpw)
    cat2 = product_category.astype(jnp.int32).reshape(NW, bpw)
    ps2 = popular_score.reshape(NW, bpw)

    mesh = plsc.VectorSubcoreMesh(core_axis_name="c", subcore_axis_name="s")

    @functools.partial(
        pl.kernel, mesh=mesh,
        compiler_params=pltpu.CompilerParams(
            needs_layout_passes=False, use_tc_tiling_on_sc=False),
        out_type=jax.ShapeDtypeStruct((3 * D // 8, B // 128, 8, 128),
                                      jnp.float32),
        scratch_types=[
            pltpu.VMEM((bpw,), jnp.int32),
            pltpu.VMEM((bpw,), jnp.int32),
            pltpu.VMEM((bpw,), jnp.float32),
            pltpu.VMEM((NBK,), jnp.float32),
            pltpu.VMEM((D * nrow, 128), jnp.int32),
            pltpu.VMEM((D * VS,), jnp.float32),
            pltpu.VMEM((D * VC,), jnp.float32),
            pltpu.VMEM((3 * D // 8, nrow, 8, 128), jnp.float32),
            pltpu.VMEM_SHARED((D * VS,), jnp.float32),
            pltpu.VMEM_SHARED((D * VC,), jnp.float32),
            pltpu.SemaphoreType.DMA,
        ],
    )
    def sc_kernel(pid_hbm, ps_hbm, cat_hbm, ttab, stab, ctab, bkt_hbm,
                  out_hbm, pid_v, cat_v, ps_v, bkt_v, tidx_v, stab_v,
                  ctab_v, outt_v, stab_sh, ctab_sh, sem_t):
        sid = lax.axis_index("s")
        wid = sid * NC + lax.axis_index("c")

        pltpu.sync_copy(pid_hbm.at[wid], pid_v)

        @plsc.parallel_loop(0, nchunk, 1, unroll=2)
        def build(i):
            off = pl.multiple_of(i * L, L)
            pid16 = pid_v[pl.ds(off, L)]
            row0 = i // 8
            col = pl.multiple_of((i % 8) * L, L)
            for c in range(D):
                tidx_v[c * nrow + row0, pl.ds(col, L)] = pid16 + (c * VT)

        copies = []
        for c in range(D):
            for j in range(nrow):
                copies.append(pltpu.async_copy(
                    ttab.at[tidx_v.at[c * nrow + j]],
                    outt_v.at[c // 8, j, c % 8], sem_t))

        pltpu.sync_copy(cat_hbm.at[wid], cat_v)
        pltpu.sync_copy(ps_hbm.at[wid], ps_v)
        @pl.when(sid == 0)
        def _():
            pltpu.sync_copy(stab, stab_sh)
            pltpu.sync_copy(ctab, ctab_sh)
        plsc.subcore_barrier()
        pltpu.sync_copy(stab_sh, stab_v)
        pltpu.sync_copy(ctab_sh, ctab_v)
        pltpu.sync_copy(bkt_hbm, bkt_v.at[pl.ds(0, NBUCK)])
        sentinel = jnp.full((L,), 2.0, jnp.float32)
        bkt_v[pl.ds(NBUCK, L)] = sentinel
        bkt_v[pl.ds(NBK - L, L)] = sentinel

        @plsc.parallel_loop(0, nchunk, 1, unroll=2)
        def chunk(i):
            off = pl.multiple_of(i * L, L)
            v = ps_v[pl.ds(off, L)]
            pos = jnp.zeros((L,), jnp.int32)
            step = NBK // 2
            while step >= 1:
                probe = pos + (step - 1)
                bv = plsc.load_gather(bkt_v, [probe])
                pos = jnp.where(bv <= v, pos + step, pos)
                step //= 2
            cats = cat_v[pl.ds(off, L)]
            bc = i // 8
            cc = pl.multiple_of((i % 8) * L, L)
            for c in range(D):
                ds_ = D + c
                dc = 2 * D + c
                outt_v[ds_ // 8, bc, ds_ % 8, pl.ds(cc, L)] = \
                    plsc.load_gather(stab_v, [pos + (c * VS)])
                outt_v[dc // 8, bc, dc % 8, pl.ds(cc, L)] = \
                    plsc.load_gather(ctab_v, [cats + (c * VC)])

        for c in copies:
            c.wait()

        pltpu.sync_copy(outt_v, out_hbm.at[:, pl.ds(wid * nrow, nrow)])

    out4 = sc_kernel(pid2, ps2, cat2, tt1, st1, ct1, buckets)
    return out4.transpose(0, 2, 1, 3).reshape(3 * D, B).T


# speedup vs baseline: 1.9232x; 1.0076x over previous
import functools

import jax
import jax.numpy as jnp
from jax import lax
from jax.experimental import pallas as pl
from jax.experimental.pallas import tpu as pltpu
from jax.experimental.pallas import tpu_sc as plsc

D = 24
NBK = 1024


def kernel(product_id, popular_score, product_category,
           title_table, scores_table, category_table, buckets):
    B = product_id.shape[0]
    VT = title_table.shape[0]
    VS = scores_table.shape[0]
    VC = category_table.shape[0]
    NBUCK = buckets.shape[0]
    info = plsc.get_sparse_core_info()
    NC, NS, L = info.num_cores, info.num_subcores, info.num_lanes
    NW = NC * NS
    bpw = B // NW
    nrow = bpw // 128
    nchunk = bpw // L

    tt1 = title_table.T.reshape(D * VT)
    st1 = scores_table.T.reshape(D * VS)
    ct1 = category_table.T.reshape(D * VC)
    pid2 = product_id.astype(jnp.int32).reshape(NW, bpw)
    cat2 = product_category.astype(jnp.int32).reshape(NW, bpw)
    ps2 = popular_score.reshape(NW, bpw)

    mesh = plsc.VectorSubcoreMesh(core_axis_name="c", subcore_axis_name="s")

    @functools.partial(
        pl.kernel, mesh=mesh,
        compiler_params=pltpu.CompilerParams(
            needs_layout_passes=False, use_tc_tiling_on_sc=False),
        out_type=jax.ShapeDtypeStruct((3 * D // 8, B // 128, 8, 128),
                                      jnp.float32),
        scratch_types=[
            pltpu.VMEM((bpw,), jnp.int32),
            pltpu.VMEM((bpw,), jnp.int32),
            pltpu.VMEM((bpw,), jnp.float32),
            pltpu.VMEM((NBK,), jnp.float32),
            pltpu.VMEM((D * nrow, 128), jnp.int32),
            pltpu.VMEM((D * VS,), jnp.float32),
            pltpu.VMEM((D * VC,), jnp.float32),
            pltpu.VMEM((3 * D // 8, nrow, 8, 128), jnp.float32),
            pltpu.VMEM_SHARED((D * VS,), jnp.float32),
            pltpu.VMEM_SHARED((D * VC,), jnp.float32),
            pltpu.SemaphoreType.DMA,
        ],
    )
    def sc_kernel(pid_hbm, ps_hbm, cat_hbm, ttab, stab, ctab, bkt_hbm,
                  out_hbm, pid_v, cat_v, ps_v, bkt_v, tidx_v, stab_v,
                  ctab_v, outt_v, stab_sh, ctab_sh, sem_t):
        sid = lax.axis_index("s")
        wid = sid * NC + lax.axis_index("c")

        pltpu.sync_copy(pid_hbm.at[wid], pid_v)

        @plsc.parallel_loop(0, nchunk, 1, unroll=2)
        def build(i):
            off = pl.multiple_of(i * L, L)
            pid16 = pid_v[pl.ds(off, L)]
            row0 = i // 8
            col = pl.multiple_of((i % 8) * L, L)
            for c in range(D):
                tidx_v[c * nrow + row0, pl.ds(col, L)] = pid16 + (c * VT)

        copies = []
        for c in range(D):
            for j in range(nrow):
                copies.append(pltpu.async_copy(
                    ttab.at[tidx_v.at[c * nrow + j]],
                    outt_v.at[c // 8, j, c % 8], sem_t))

        pltpu.sync_copy(cat_hbm.at[wid], cat_v)
        pltpu.sync_copy(ps_hbm.at[wid], ps_v)
        @pl.when(sid == 0)
        def _():
            pltpu.sync_copy(stab, stab_sh)
            pltpu.sync_copy(ctab, ctab_sh)
        plsc.subcore_barrier()
        pltpu.sync_copy(stab_sh, stab_v)
        pltpu.sync_copy(ctab_sh, ctab_v)
        pltpu.sync_copy(bkt_hbm, bkt_v.at[pl.ds(0, NBUCK)])
        sentinel = jnp.full((L,), 2.0, jnp.float32)
        bkt_v[pl.ds(NBUCK, L)] = sentinel
        bkt_v[pl.ds(NBK - L, L)] = sentinel

        @plsc.parallel_loop(0, nchunk, 1, unroll=4)
        def chunk(i):
            off = pl.multiple_of(i * L, L)
            v = ps_v[pl.ds(off, L)]
            pos = jnp.zeros((L,), jnp.int32)
            step = NBK // 2
            while step >= 1:
                probe = pos + (step - 1)
                bv = plsc.load_gather(bkt_v, [probe])
                pos = jnp.where(bv <= v, pos + step, pos)
                step //= 2
            cats = cat_v[pl.ds(off, L)]
            bc = i // 8
            cc = pl.multiple_of((i % 8) * L, L)
            for c in range(D):
                ds_ = D + c
                dc = 2 * D + c
                outt_v[ds_ // 8, bc, ds_ % 8, pl.ds(cc, L)] = \
                    plsc.load_gather(stab_v, [pos + (c * VS)])
                outt_v[dc // 8, bc, dc % 8, pl.ds(cc, L)] = \
                    plsc.load_gather(ctab_v, [cats + (c * VC)])

        nt = D // 8
        out_sc = pltpu.async_copy(
            outt_v.at[pl.ds(nt, 2 * nt)],
            out_hbm.at[pl.ds(nt, 2 * nt), pl.ds(wid * nrow, nrow)], sem_t)
        for c in copies:
            c.wait()
        pltpu.sync_copy(outt_v.at[pl.ds(0, nt)],
                        out_hbm.at[pl.ds(0, nt), pl.ds(wid * nrow, nrow)])
        out_sc.wait()

    out4 = sc_kernel(pid2, ps2, cat2, tt1, st1, ct1, buckets)
    return out4.transpose(0, 2, 1, 3).reshape(3 * D, B).T
